# Initial kernel scaffold; baseline (speedup 1.0000x reference)
import jax
import jax.numpy as jnp
from jax.experimental import pallas as pl


def kernel(y_pred, y_true, x_input, edge_attr, x_mean, x_std, y_mean, y_std, edge_mean, edge_std, edge_index):
    raise NotImplementedError("write your pallas kernel here")


# speedup vs baseline: 7.2801x; 7.2801x over previous
import math

import jax
import jax.numpy as jnp
from jax import lax
from jax.experimental import pallas as pl
from jax.experimental.pallas import tpu as pltpu
from jax.experimental.pallas import tpu_sc as plsc

N = 100000
E = N * 32

NC = 2
NS = 16
NW = NC * NS

W = 1024
NWIN_TOT = E // W
BASE_WIN = NWIN_TOT // NW
EXTRA = NWIN_TOT - BASE_WIN * NW
NODE_CHUNK = 6256
NPAD = NODE_CHUNK * NS
NROW = NPAD * 2 // 128
EROW = E * 2 // 128


def _sel(parity):
  l = lax.broadcasted_iota(jnp.int32, (128, 64), 0)
  c = lax.broadcasted_iota(jnp.int32, (128, 64), 1)
  return (l == 2 * c + parity).astype(jnp.float32)


def _dot(a, s):
  return lax.dot_general(a, s, (((1,), (0,)), ((), ())),
                         precision=lax.Precision.HIGHEST,
                         preferred_element_type=jnp.float32)


def _node_prep_body(yp2d, yt2d, x2d, ystd_t, ymean_t, xstd_t, xmean_t,
                    e_ref, f_ref, p_ref, q_ref, mse_ref):
  s0 = _sel(0)
  s1 = _sel(1)
  ypn = yp2d[:, :] * ystd_t[:, :] + ymean_t[:, :]
  vm = _dot(ypn, s0)
  va = _dot(ypn, s1) * (math.pi / 180.0)
  e_ref[:, :] = vm * jnp.cos(va)
  f_ref[:, :] = vm * jnp.sin(va)
  xn = (x2d[:, :] * xstd_t[:, :] + xmean_t[:, :]) * (1.0 / 100.0)
  p_ref[:, :] = _dot(xn, s0)
  q_ref[:, :] = _dot(xn, s1)
  d = (yp2d[:, :] - yt2d[:, :]) * ystd_t[:, :]
  mse_ref[:, :] = jnp.sum(d * d).reshape(1, 1)


def _edge_prep_body(a2d, estd_t, emean_t, g_ref, b_ref):
  s0 = _sel(0)
  s1 = _sel(1)
  apu = a2d[:, :] * estd_t[:, :] + emean_t[:, :]
  r = _dot(apu, s0)
  x = _dot(apu, s1)
  inv = 1.0 / (r * r + x * x)
  g_ref[:, :] = r * inv
  b_ref[:, :] = -(x * inv)


def _edge_body(e_hbm, f_hbm, idxi_hbm, idxj_hbm, g_hbm, b_hbm, z_hbm,
               out_hbm,
               e_sh, f_sh, aggp_sh, aggq_sh,
               idxi_v, idxj_v, g_v, b_v, ei_v, fi_v, ej_v, fj_v, p_v, q_v,
               buf_v, sem_g, sem_s):
  c = lax.axis_index("c")
  s = lax.axis_index("s")
  wid = s * NC + c

  row0 = s * NODE_CHUNK
  sl = pl.ds(row0, NODE_CHUNK)
  pltpu.sync_copy(z_hbm, buf_v)
  pltpu.sync_copy(buf_v, aggp_sh.at[sl])
  pltpu.sync_copy(buf_v, aggq_sh.at[sl])
  pltpu.sync_copy(e_hbm.at[sl], buf_v)
  pltpu.sync_copy(buf_v, e_sh.at[sl])
  pltpu.sync_copy(f_hbm.at[sl], buf_v)
  pltpu.sync_copy(buf_v, f_sh.at[sl])
  plsc.subcore_barrier()

  nwin = jnp.where(wid < EXTRA, BASE_WIN + 1, BASE_WIN)
  win_start = wid * BASE_WIN + jnp.minimum(wid, EXTRA)

  def window(i, _):
    base = pl.multiple_of((win_start + i) * W, W)
    esl = pl.ds(base, W)
    pltpu.sync_copy(idxi_hbm.at[esl], idxi_v)
    pltpu.sync_copy(idxj_hbm.at[esl], idxj_v)
    pltpu.sync_copy(g_hbm.at[esl], g_v)
    pltpu.sync_copy(b_hbm.at[esl], b_v)

    d1 = pltpu.async_copy(e_sh.at[idxi_v], ei_v, sem_g)
    d2 = pltpu.async_copy(f_sh.at[idxi_v], fi_v, sem_g)
    d3 = pltpu.async_copy(e_sh.at[idxj_v], ej_v, sem_g)
    d4 = pltpu.async_copy(f_sh.at[idxj_v], fj_v, sem_g)
    d1.wait()
    d2.wait()
    d3.wait()
    d4.wait()

    for t in range(W // 16):
      gsl = pl.ds(t * 16, 16)
      ei = ei_v[gsl]
      fi = fi_v[gsl]
      ej = ej_v[gsl]
      fj = fj_v[gsl]
      g = g_v[gsl]
      b = b_v[gsl]
      a = ei * ej + fi * fj - ei * ei - fi * fi
      bc = fi * ej - ei * fj
      p_v[gsl] = g * a + b * bc
      q_v[gsl] = g * bc - b * a

    s1 = pltpu.async_copy(p_v, aggp_sh.at[idxi_v], sem_s, add=True)
    s2 = pltpu.async_copy(q_v, aggq_sh.at[idxi_v], sem_s, add=True)
    s1.wait()
    s2.wait()
    return ()

  lax.fori_loop(0, nwin, window, ())

  plsc.subcore_barrier()
  off_p = (c * 2 + 0) * NPAD + row0
  off_q = (c * 2 + 1) * NPAD + row0
  pltpu.sync_copy(aggp_sh.at[sl], buf_v)
  pltpu.sync_copy(buf_v, out_hbm.at[pl.ds(off_p, NODE_CHUNK)])
  pltpu.sync_copy(aggq_sh.at[sl], buf_v)
  pltpu.sync_copy(buf_v, out_hbm.at[pl.ds(off_q, NODE_CHUNK)])


def _final_body(p2d, q2d, ap0, aq0, ap1, aq1, mse_ref, out_ref):
  dp = p2d[:, :] - (ap0[:, :] + ap1[:, :])
  dq = q2d[:, :] - (aq0[:, :] + aq1[:, :])
  phys_sum = jnp.sum(dp * dp + dq * dq)
  mse = mse_ref[0, 0] * (1.0 / (2.0 * N))
  phys = phys_sum * (1.0 / N)
  out_ref[:, :] = (0.9 * mse + (1.0 - 0.9) * 0.02 * phys).reshape(1, 1)


def kernel(y_pred, y_true, x_input, edge_attr, x_mean, x_std, y_mean, y_std,
           edge_mean, edge_std, edge_index):
  f32 = jnp.float32
  pad = NPAD - N
  yp2d = jnp.pad(y_pred, ((0, pad), (0, 0))).reshape(NROW, 128)
  yt2d = jnp.pad(y_true, ((0, pad), (0, 0))).reshape(NROW, 128)
  x2d = jnp.pad(x_input[:, 0:2], ((0, pad), (0, 0))).reshape(NROW, 128)
  a2d = edge_attr.reshape(EROW, 128)

  ystd_t = jnp.tile(y_std.reshape(2), 64).reshape(1, 128)
  ymean_t = jnp.tile(y_mean.reshape(2), 64).reshape(1, 128)
  xstd_t = jnp.tile(x_std[0, 0:2], 64).reshape(1, 128)
  xmean_t = jnp.tile(x_mean[0, 0:2], 64).reshape(1, 128)
  estd_t = jnp.tile(edge_std, 64).reshape(1, 128)
  emean_t = jnp.tile(edge_mean, 64).reshape(1, 128)

  e2d, f2d, p2d, q2d, mse_sum = pl.pallas_call(
      _node_prep_body,
      out_shape=(
          jax.ShapeDtypeStruct((NROW, 64), f32),
          jax.ShapeDtypeStruct((NROW, 64), f32),
          jax.ShapeDtypeStruct((NROW, 64), f32),
          jax.ShapeDtypeStruct((NROW, 64), f32),
          jax.ShapeDtypeStruct((1, 1), f32),
      ),
  )(yp2d, yt2d, x2d, ystd_t, ymean_t, xstd_t, xmean_t)

  gblk = 2000
  g2d, b2d = pl.pallas_call(
      _edge_prep_body,
      grid=(EROW // gblk,),
      in_specs=[
          pl.BlockSpec((gblk, 128), lambda i: (i, 0)),
          pl.BlockSpec((1, 128), lambda i: (0, 0)),
          pl.BlockSpec((1, 128), lambda i: (0, 0)),
      ],
      out_specs=[
          pl.BlockSpec((gblk, 64), lambda i: (i, 0)),
          pl.BlockSpec((gblk, 64), lambda i: (i, 0)),
      ],
      out_shape=(
          jax.ShapeDtypeStruct((EROW, 64), f32),
          jax.ShapeDtypeStruct((EROW, 64), f32),
      ),
  )(a2d, estd_t, emean_t)

  e_flat = e2d.reshape(NPAD)
  f_flat = f2d.reshape(NPAD)
  g_flat = g2d.reshape(E)
  b_flat = b2d.reshape(E)
  zeros1 = jnp.zeros((NODE_CHUNK,), f32)

  mesh = plsc.VectorSubcoreMesh(core_axis_name="c", subcore_axis_name="s")
  parts = pl.kernel(
      _edge_body,
      out_type=jax.ShapeDtypeStruct((4 * NPAD,), f32),
      mesh=mesh,
      scratch_types=[
          pltpu.VMEM_SHARED((NPAD,), f32),
          pltpu.VMEM_SHARED((NPAD,), f32),
          pltpu.VMEM_SHARED((NPAD,), f32),
          pltpu.VMEM_SHARED((NPAD,), f32),
          pltpu.VMEM((W,), jnp.int32),
          pltpu.VMEM((W,), jnp.int32),
          pltpu.VMEM((W,), f32),
          pltpu.VMEM((W,), f32),
          pltpu.VMEM((W,), f32),
          pltpu.VMEM((W,), f32),
          pltpu.VMEM((W,), f32),
          pltpu.VMEM((W,), f32),
          pltpu.VMEM((W,), f32),
          pltpu.VMEM((W,), f32),
          pltpu.VMEM((NODE_CHUNK,), f32),
          pltpu.SemaphoreType.DMA,
          pltpu.SemaphoreType.DMA,
      ],
  )(e_flat, f_flat, edge_index[0], edge_index[1], g_flat, b_flat, zeros1)

  parts4 = parts.reshape(4, NPAD // 128, 128)
  out = pl.pallas_call(
      _final_body,
      out_shape=jax.ShapeDtypeStruct((1, 1), f32),
  )(p2d.reshape(NPAD // 128, 128), q2d.reshape(NPAD // 128, 128),
    parts4[0], parts4[1], parts4[2], parts4[3], mse_sum)
  return out[0, 0]
